# baseline (speedup 1.0000x reference)
import jax
import jax.numpy as jnp
import numpy as np
from jax import lax
from jax.experimental import pallas as pl
from jax.experimental.pallas import tpu as pltpu
from jax.experimental.pallas import tpu_sc as plsc

N = 10000
E = 160000
D = 128

NC = 2
NS = 16
NT = NC * NS
L = 16

EDGES_PER_TILE = E // NT
CHUNK = 128
NCHUNK = (EDGES_PER_TILE + CHUNK - 1) // CHUNK

N_PAD = 10240
ROWS_PER_TILE = N_PAD // NS
ZROWS = 40


def _sc_body(feat_hbm, src_hbm, dst_hbm, out_hbm,
             accum, src2d, dst2d, rows_a, rows_b, zbuf,
             gsem_a, gsem_b):
    c = lax.axis_index("c")
    s = lax.axis_index("s")
    t = c * NS + s
    row0 = s * ROWS_PER_TILE

    z16 = jnp.zeros((L,), jnp.float32)

    def zfill(i, carry):
        for k in range(D // L):
            zbuf[i, pl.ds(k * L, L)] = z16
        return carry
    lax.fori_loop(0, ZROWS, zfill, 0)

    def zero_own_rows():
        def zcopy(k, carry):
            pltpu.sync_copy(zbuf, accum.at[pl.ds(row0 + k * ZROWS, ZROWS)])
            return carry
        lax.fori_loop(0, ROWS_PER_TILE // ZROWS, zcopy, 0)

    zero_own_rows()
    plsc.subcore_barrier()

    for r in range(3):
        pltpu.sync_copy(src_hbm.at[r, t], src2d)
        pltpu.sync_copy(dst_hbm.at[r, t], dst2d)

        def start_gather(j, buf, sem):
            pltpu.async_copy(feat_hbm.at[src2d.at[j]], buf, sem)

        def wait_gather(buf, sem):
            pltpu.make_async_copy(feat_hbm.at[src2d.at[0]], buf, sem).wait()

        def scatter(buf, j):
            pltpu.sync_copy(buf, accum.at[dst2d.at[j]], add=True)

        start_gather(0, rows_a, gsem_a)

        def slot(j, rows_c, sem_c, rows_n, sem_n):
            wait_gather(rows_c, sem_c)

            @pl.when(j + 1 < NCHUNK)
            def _():
                start_gather(j + 1, rows_n, sem_n)
            scatter(rows_c, j)

        def chunk_pair(i, carry):
            jj = 2 * i
            slot(jj, rows_a, gsem_a, rows_b, gsem_b)
            slot(jj + 1, rows_b, gsem_b, rows_a, gsem_a)
            return carry
        lax.fori_loop(0, NCHUNK // 2, chunk_pair, 0)

        plsc.subcore_barrier()
        oc = c * 3 + r

        def fcopy(k, carry, oc=oc):
            sl = pl.ds(row0 + k * ZROWS, ZROWS)
            pltpu.sync_copy(accum.at[sl], out_hbm.at[oc, sl])
            return carry
        lax.fori_loop(0, ROWS_PER_TILE // ZROWS, fcopy, 0)
        if r < 2:
            zero_own_rows()
        plsc.subcore_barrier()


NPAD_EDGES = NCHUNK * CHUNK - EDGES_PER_TILE

_DUMP_ROWS = (N + (np.arange(NT * NPAD_EDGES, dtype=np.int32)
                   % (N_PAD - N)).reshape(NT, NPAD_EDGES))


def _pad_idx(ei, pad_block):
    x = ei.reshape(NT, EDGES_PER_TILE)
    x = jnp.concatenate([x, pad_block], axis=1)
    return x.reshape(NT, NCHUNK, CHUNK)


_SRC_PAD = np.arange(NT * NPAD_EDGES, dtype=np.int32).reshape(NT, NPAD_EDGES) % N


def _sc_accumulate(features, ei0, ei1, ei2):
    spad = jnp.asarray(_SRC_PAD)
    dpad = jnp.asarray(_DUMP_ROWS)
    src = jnp.stack([_pad_idx(e[0], spad) for e in (ei0, ei1, ei2)])
    dst = jnp.stack([_pad_idx(e[1], dpad) for e in (ei0, ei1, ei2)])
    mesh = plsc.VectorSubcoreMesh(core_axis_name="c", subcore_axis_name="s",
                                  num_cores=NC, num_subcores=NS)
    return pl.kernel(
        _sc_body,
        out_type=jax.ShapeDtypeStruct((2 * 3, N_PAD, D), jnp.float32),
        mesh=mesh,
        scratch_types=[
            pltpu.VMEM_SHARED((N_PAD, D), jnp.float32),
            pltpu.VMEM((NCHUNK, CHUNK), jnp.int32),
            pltpu.VMEM((NCHUNK, CHUNK), jnp.int32),
            pltpu.VMEM((CHUNK, D), jnp.float32),
            pltpu.VMEM((CHUNK, D), jnp.float32),
            pltpu.VMEM((ZROWS, D), jnp.float32),
            pltpu.SemaphoreType.DMA,
            pltpu.SemaphoreType.DMA,
        ],
    )(features, src, dst)


BN = 1000


def _tc_body(p_ref, x_ref, w_ref, o_ref):
    a0 = p_ref[0] + p_ref[3]
    a1 = p_ref[1] + p_ref[4]
    a2 = p_ref[2] + p_ref[5]
    acc = jnp.dot(a0, w_ref[0], preferred_element_type=jnp.float32)
    acc = acc + jnp.dot(a1, w_ref[1], preferred_element_type=jnp.float32)
    acc = acc - jnp.dot(a2, w_ref[2], preferred_element_type=jnp.float32)
    acc = acc + jnp.dot(x_ref[...], w_ref[3], preferred_element_type=jnp.float32)
    o_ref[...] = jnp.maximum(acc, 0.0)


def _tc_finish(partials, features, wstk):
    return pl.pallas_call(
        _tc_body,
        out_shape=jax.ShapeDtypeStruct((N, D), jnp.float32),
        grid=(N // BN,),
        in_specs=[
            pl.BlockSpec((6, BN, D), lambda i: (0, i, 0)),
            pl.BlockSpec((BN, D), lambda i: (i, 0)),
            pl.BlockSpec((4, D, D), lambda i: (0, 0, 0)),
        ],
        out_specs=pl.BlockSpec((BN, D), lambda i: (i, 0)),
    )(partials, features, wstk)


def kernel(features, edge_index_rel0, edge_index_rel1, edge_index_rel2,
           W_rel0, W_rel1, W_rel2, W_self):
    partials = _sc_accumulate(features, edge_index_rel0, edge_index_rel1,
                              edge_index_rel2)
    wstk = jnp.stack([W_rel0, W_rel1, W_rel2, W_self])
    return _tc_finish(partials, features, wstk)


# speedup vs baseline: 1.0209x; 1.0209x over previous
import jax
import jax.numpy as jnp
import numpy as np
from jax import lax
from jax.experimental import pallas as pl
from jax.experimental.pallas import tpu as pltpu
from jax.experimental.pallas import tpu_sc as plsc

N = 10000
E = 160000
D = 128

NC = 2
NS = 16
NT = NC * NS
L = 16

EDGES_PER_TILE = E // NT
CHUNK = 128
NCHUNK = (EDGES_PER_TILE + CHUNK - 1) // CHUNK

N_PAD = 10240
ROWS_PER_TILE = N_PAD // NS
ZROWS = 40


def _sc_body(feat_hbm, src_hbm, dst_hbm, zeros_hbm, out_hbm,
             accum, src2d, dst2d, rows_a, rows_b,
             gsem_a, gsem_b):
    c = lax.axis_index("c")
    s = lax.axis_index("s")
    t = c * NS + s
    row0 = s * ROWS_PER_TILE

    def zero_own_rows():
        pltpu.sync_copy(zeros_hbm, accum.at[pl.ds(row0, ROWS_PER_TILE)])

    zero_own_rows()
    plsc.subcore_barrier()

    for r in range(3):
        pltpu.sync_copy(src_hbm.at[r, t], src2d)
        pltpu.sync_copy(dst_hbm.at[r, t], dst2d)

        def start_gather(j, buf, sem):
            pltpu.async_copy(feat_hbm.at[src2d.at[j]], buf, sem)

        def wait_gather(buf, sem):
            pltpu.make_async_copy(feat_hbm.at[src2d.at[0]], buf, sem).wait()

        def scatter(buf, j):
            pltpu.sync_copy(buf, accum.at[dst2d.at[j]], add=True)

        start_gather(0, rows_a, gsem_a)

        def slot(j, rows_c, sem_c, rows_n, sem_n):
            wait_gather(rows_c, sem_c)

            @pl.when(j + 1 < NCHUNK)
            def _():
                start_gather(j + 1, rows_n, sem_n)
            scatter(rows_c, j)

        def chunk_pair(i, carry):
            jj = 2 * i
            slot(jj, rows_a, gsem_a, rows_b, gsem_b)
            slot(jj + 1, rows_b, gsem_b, rows_a, gsem_a)
            return carry
        lax.fori_loop(0, NCHUNK // 2, chunk_pair, 0)

        plsc.subcore_barrier()
        oc = c * 3 + r
        sl = pl.ds(row0, ROWS_PER_TILE)
        pltpu.sync_copy(accum.at[sl], out_hbm.at[oc, sl])
        if r < 2:
            zero_own_rows()
        plsc.subcore_barrier()


NPAD_EDGES = NCHUNK * CHUNK - EDGES_PER_TILE

_DUMP_ROWS = (N + (np.arange(NT * NPAD_EDGES, dtype=np.int32)
                   % (N_PAD - N)).reshape(NT, NPAD_EDGES))


def _pad_idx(ei, pad_block):
    x = ei.reshape(NT, EDGES_PER_TILE)
    x = jnp.concatenate([x, pad_block], axis=1)
    return x.reshape(NT, NCHUNK, CHUNK)


_SRC_PAD = np.arange(NT * NPAD_EDGES, dtype=np.int32).reshape(NT, NPAD_EDGES) % N


def _sc_accumulate(features, ei0, ei1, ei2):
    spad = jnp.asarray(_SRC_PAD)
    dpad = jnp.asarray(_DUMP_ROWS)
    src = jnp.stack([_pad_idx(e[0], spad) for e in (ei0, ei1, ei2)])
    dst = jnp.stack([_pad_idx(e[1], dpad) for e in (ei0, ei1, ei2)])
    mesh = plsc.VectorSubcoreMesh(core_axis_name="c", subcore_axis_name="s",
                                  num_cores=NC, num_subcores=NS)
    return pl.kernel(
        _sc_body,
        out_type=jax.ShapeDtypeStruct((2 * 3, N_PAD, D), jnp.float32),
        mesh=mesh,
        scratch_types=[
            pltpu.VMEM_SHARED((N_PAD, D), jnp.float32),
            pltpu.VMEM((NCHUNK, CHUNK), jnp.int32),
            pltpu.VMEM((NCHUNK, CHUNK), jnp.int32),
            pltpu.VMEM((CHUNK, D), jnp.float32),
            pltpu.VMEM((CHUNK, D), jnp.float32),
            pltpu.SemaphoreType.DMA,
            pltpu.SemaphoreType.DMA,
        ],
    )(features, src, dst, jnp.zeros((ROWS_PER_TILE, D), jnp.float32))


BN = 1000


def _tc_body(p_ref, x_ref, w_ref, o_ref):
    a0 = p_ref[0] + p_ref[3]
    a1 = p_ref[1] + p_ref[4]
    a2 = p_ref[2] + p_ref[5]
    acc = jnp.dot(a0, w_ref[0], preferred_element_type=jnp.float32)
    acc = acc + jnp.dot(a1, w_ref[1], preferred_element_type=jnp.float32)
    acc = acc - jnp.dot(a2, w_ref[2], preferred_element_type=jnp.float32)
    acc = acc + jnp.dot(x_ref[...], w_ref[3], preferred_element_type=jnp.float32)
    o_ref[...] = jnp.maximum(acc, 0.0)


def _tc_finish(partials, features, wstk):
    return pl.pallas_call(
        _tc_body,
        out_shape=jax.ShapeDtypeStruct((N, D), jnp.float32),
        grid=(N // BN,),
        in_specs=[
            pl.BlockSpec((6, BN, D), lambda i: (0, i, 0)),
            pl.BlockSpec((BN, D), lambda i: (i, 0)),
            pl.BlockSpec((4, D, D), lambda i: (0, 0, 0)),
        ],
        out_specs=pl.BlockSpec((BN, D), lambda i: (i, 0)),
    )(partials, features, wstk)


def kernel(features, edge_index_rel0, edge_index_rel1, edge_index_rel2,
           W_rel0, W_rel1, W_rel2, W_self):
    partials = _sc_accumulate(features, edge_index_rel0, edge_index_rel1,
                              edge_index_rel2)
    wstk = jnp.stack([W_rel0, W_rel1, W_rel2, W_self])
    return _tc_finish(partials, features, wstk)


# speedup vs baseline: 1.0496x; 1.0282x over previous
import jax
import jax.numpy as jnp
import numpy as np
from jax import lax
from jax.experimental import pallas as pl
from jax.experimental.pallas import tpu as pltpu
from jax.experimental.pallas import tpu_sc as plsc

N = 10000
E = 160000
D = 128

NC = 2
NS = 16
NT = NC * NS
L = 16

EDGES_PER_TILE = E // NT
CHUNK = 128
NCHUNK = (EDGES_PER_TILE + CHUNK - 1) // CHUNK

N_PAD = 10240
ROWS_PER_TILE = N_PAD // NS
ZROWS = 40


def _sc_body(feat_hbm, src_hbm, dst_hbm, out_hbm,
             accum, src2d, dst2d, rows_a, rows_b, zbuf,
             gsem_a, gsem_b):
    c = lax.axis_index("c")
    s = lax.axis_index("s")
    t = c * NS + s
    row0 = s * ROWS_PER_TILE

    z16 = jnp.zeros((L,), jnp.float32)

    def zfill(i, carry):
        for k in range(D // L):
            zbuf[i, pl.ds(k * L, L)] = z16
        return carry
    lax.fori_loop(0, ZROWS, zfill, 0)

    def zero_own_rows():
        def zcopy(k, carry):
            pltpu.sync_copy(zbuf, accum.at[pl.ds(row0 + k * ZROWS, ZROWS)])
            return carry
        lax.fori_loop(0, ROWS_PER_TILE // ZROWS, zcopy, 0)

    zero_own_rows()
    plsc.subcore_barrier()

    for r in range(3):
        pltpu.sync_copy(src_hbm.at[r, t], src2d)
        pltpu.sync_copy(dst_hbm.at[r, t], dst2d)

        def start_gather(j, buf, sem):
            pltpu.async_copy(feat_hbm.at[src2d.at[j]], buf, sem)

        def wait_gather(buf, sem):
            pltpu.make_async_copy(feat_hbm.at[src2d.at[0]], buf, sem).wait()

        def scatter(buf, j):
            pltpu.sync_copy(buf, accum.at[dst2d.at[j]], add=True)

        start_gather(0, rows_a, gsem_a)

        def slot(j, rows_c, sem_c, rows_n, sem_n):
            wait_gather(rows_c, sem_c)

            @pl.when(j + 1 < NCHUNK)
            def _():
                start_gather(j + 1, rows_n, sem_n)
            scatter(rows_c, j)

        def chunk_pair(i, carry):
            jj = 2 * i
            slot(jj, rows_a, gsem_a, rows_b, gsem_b)
            slot(jj + 1, rows_b, gsem_b, rows_a, gsem_a)
            return carry
        lax.fori_loop(0, NCHUNK // 2, chunk_pair, 0)

        plsc.subcore_barrier()
        oc = c * 3 + r
        sl = pl.ds(row0, ROWS_PER_TILE)
        pltpu.sync_copy(accum.at[sl], out_hbm.at[oc, sl])
        if r < 2:
            zero_own_rows()
        plsc.subcore_barrier()


NPAD_EDGES = NCHUNK * CHUNK - EDGES_PER_TILE

_DUMP_ROWS = (N + (np.arange(NT * NPAD_EDGES, dtype=np.int32)
                   % (N_PAD - N)).reshape(NT, NPAD_EDGES))


def _pad_idx(ei, pad_block):
    x = ei.reshape(NT, EDGES_PER_TILE)
    x = jnp.concatenate([x, pad_block], axis=1)
    return x.reshape(NT, NCHUNK, CHUNK)


_SRC_PAD = np.arange(NT * NPAD_EDGES, dtype=np.int32).reshape(NT, NPAD_EDGES) % N


def _sc_accumulate(features, ei0, ei1, ei2):
    spad = jnp.asarray(_SRC_PAD)
    dpad = jnp.asarray(_DUMP_ROWS)
    src = jnp.stack([_pad_idx(e[0], spad) for e in (ei0, ei1, ei2)])
    dst = jnp.stack([_pad_idx(e[1], dpad) for e in (ei0, ei1, ei2)])
    mesh = plsc.VectorSubcoreMesh(core_axis_name="c", subcore_axis_name="s",
                                  num_cores=NC, num_subcores=NS)
    return pl.kernel(
        _sc_body,
        out_type=jax.ShapeDtypeStruct((2 * 3, N_PAD, D), jnp.float32),
        mesh=mesh,
        scratch_types=[
            pltpu.VMEM_SHARED((N_PAD, D), jnp.float32),
            pltpu.VMEM((NCHUNK, CHUNK), jnp.int32),
            pltpu.VMEM((NCHUNK, CHUNK), jnp.int32),
            pltpu.VMEM((CHUNK, D), jnp.float32),
            pltpu.VMEM((CHUNK, D), jnp.float32),
            pltpu.VMEM((ZROWS, D), jnp.float32),
            pltpu.SemaphoreType.DMA,
            pltpu.SemaphoreType.DMA,
        ],
    )(features, src, dst)


BN = 1000


def _tc_body(p_ref, x_ref, w_ref, o_ref):
    a0 = p_ref[0] + p_ref[3]
    a1 = p_ref[1] + p_ref[4]
    a2 = p_ref[2] + p_ref[5]
    acc = jnp.dot(a0, w_ref[0], preferred_element_type=jnp.float32)
    acc = acc + jnp.dot(a1, w_ref[1], preferred_element_type=jnp.float32)
    acc = acc - jnp.dot(a2, w_ref[2], preferred_element_type=jnp.float32)
    acc = acc + jnp.dot(x_ref[...], w_ref[3], preferred_element_type=jnp.float32)
    o_ref[...] = jnp.maximum(acc, 0.0)


def _tc_finish(partials, features, wstk):
    return pl.pallas_call(
        _tc_body,
        out_shape=jax.ShapeDtypeStruct((N, D), jnp.float32),
        grid=(N // BN,),
        in_specs=[
            pl.BlockSpec((6, BN, D), lambda i: (0, i, 0)),
            pl.BlockSpec((BN, D), lambda i: (i, 0)),
            pl.BlockSpec((4, D, D), lambda i: (0, 0, 0)),
        ],
        out_specs=pl.BlockSpec((BN, D), lambda i: (i, 0)),
    )(partials, features, wstk)


def kernel(features, edge_index_rel0, edge_index_rel1, edge_index_rel2,
           W_rel0, W_rel1, W_rel2, W_self):
    partials = _sc_accumulate(features, edge_index_rel0, edge_index_rel1,
                              edge_index_rel2)
    wstk = jnp.stack([W_rel0, W_rel1, W_rel2, W_self])
    return _tc_finish(partials, features, wstk)
